# speedup vs baseline: 7.7424x; 7.7424x over previous
import functools

import jax
import jax.numpy as jnp
from jax import lax
from jax.experimental import pallas as pl
from jax.experimental.pallas import tpu as pltpu
from jax.experimental.pallas import tpu_sc as plsc

_N_NODES = 100000
_N_REGIONS = 8192
_N_BOUNDARY = 50000
_B = 64

_BLK = 5000
_NBLK = _N_NODES // _BLK

_CH = 400
_NCHUNK = _N_BOUNDARY // _CH
_NW = 32
_CHUNKS_PER_W = -(-_NCHUNK // _NW)


@functools.lru_cache(maxsize=1)
def _build_hist():
    mesh = plsc.VectorSubcoreMesh(core_axis_name="c", subcore_axis_name="s")

    @functools.partial(
        pl.kernel,
        mesh=mesh,
        out_type=jax.ShapeDtypeStruct((2, _N_NODES), jnp.float32),
        scratch_types=[
            pltpu.VMEM((_CH,), jnp.int32),
            pltpu.VMEM((_CH,), jnp.float32),
            pltpu.VMEM_SHARED((_N_NODES,), jnp.float32),
        ],
    )
    def hist_kernel(bn_hbm, zero_hbm, out_hbm, idx_v, ones_v, hist_sh):
        c = lax.axis_index("c")
        s = lax.axis_index("s")
        wid = c * 16 + s

        for i in range(_CH // 16):
            ones_v[pl.ds(i * 16, 16)] = jnp.ones((16,), jnp.float32)

        @pl.when(s == 0)
        def _():
            pltpu.sync_copy(zero_hbm, hist_sh)

        plsc.subcore_barrier()

        for k in range(_CHUNKS_PER_W):
            cid = wid + _NW * k

            @pl.when(cid < _NCHUNK)
            def _():
                pltpu.sync_copy(bn_hbm.at[pl.ds(cid * _CH, _CH)], idx_v)
                pltpu.sync_copy(ones_v, hist_sh.at[idx_v], add=True)

        plsc.subcore_barrier()

        @pl.when(s == 0)
        def _():
            pltpu.sync_copy(hist_sh, out_hbm.at[c])

    return hist_kernel


def _hist(boundary_nodes, zeros):
    return _build_hist()(boundary_nodes, zeros)


def _tc_body(nbi_ref, p0_ref, p1_ref, emb_ref, rbi_ref, reg_ref,
             ws1_ref, bs1_ref, ws2_ref, bs2_ref, wb1_ref, bb1_ref,
             wb2_ref, bb2_ref, wv1a_ref, wv1b_ref, bv1_ref, wv2_ref, bv2_ref,
             out_ref, acc_n, cnt_n):
    i = pl.program_id(0)

    @pl.when(i == 0)
    def _():
        acc_n[:, :] = jnp.zeros_like(acc_n)
        cnt_n[:, :] = jnp.zeros_like(cnt_n)

    nbi = nbi_ref[0]
    cnt = p0_ref[0] + p1_ref[0]
    seg = lax.broadcasted_iota(jnp.int32, (_B, _BLK), 0)
    m = jnp.where(seg == nbi, cnt, 0.0)
    acc_n[:, :] += jnp.dot(m, emb_ref[:, :], preferred_element_type=jnp.float32)
    cnt_n[:, :] += jnp.sum(m, axis=1, keepdims=True)

    @pl.when(i == _NBLK - 1)
    def _():
        rbi = rbi_ref[0]
        segr = lax.broadcasted_iota(jnp.int32, (_B, _N_REGIONS), 0)
        mr = (segr == rbi).astype(jnp.float32)
        acc_r = jnp.dot(mr, reg_ref[:, :], preferred_element_type=jnp.float32)
        cnt_r = jnp.sum(mr, axis=1, keepdims=True)

        region_mean = acc_r / jnp.maximum(cnt_r, 1.0)
        h = jnp.maximum(
            jnp.dot(region_mean, ws1_ref[:, :], preferred_element_type=jnp.float32)
            + bs1_ref[:, :], 0.0)
        gs = jnp.maximum(
            jnp.dot(h, ws2_ref[:, :], preferred_element_type=jnp.float32)
            + bs2_ref[:, :], 0.0)

        bmean = acc_n[:, :] / jnp.maximum(cnt_n[:, :], 1.0)
        hb = jnp.maximum(
            jnp.dot(bmean, wb1_ref[:, :], preferred_element_type=jnp.float32)
            + bb1_ref[:, :], 0.0)
        binfo = (jnp.dot(hb, wb2_ref[:, :], preferred_element_type=jnp.float32)
                 + bb2_ref[:, :])

        hv = jnp.maximum(
            jnp.dot(gs, wv1a_ref[:, :], preferred_element_type=jnp.float32)
            + jnp.dot(binfo, wv1b_ref[:, :], preferred_element_type=jnp.float32)
            + bv1_ref[:, :], 0.0)
        out_ref[:, :] = (jnp.dot(hv, wv2_ref[:, :],
                                 preferred_element_type=jnp.float32)
                         + bv2_ref[:, :])


def _full_spec(arr):
    nd = arr.ndim
    return pl.BlockSpec(arr.shape, lambda i: (0,) * nd)


def _tc_forward(parts, node_embeddings, region_embeddings,
                node_batch_idx, region_batch_idx,
                W_s1, b_s1, W_s2, b_s2, W_b1, b_b1, W_b2, b_b2,
                W_v1, b_v1, W_v2, b_v2):
    parts3 = parts.reshape(2 * _NBLK, 1, _BLK)
    nbi3 = node_batch_idx.astype(jnp.int32).reshape(_NBLK, 1, _BLK)
    rbi3 = region_batch_idx.astype(jnp.int32).reshape(1, 1, _N_REGIONS)
    wv1a = W_v1[:128]
    wv1b = W_v1[128:]
    bs1 = b_s1.reshape(1, -1)
    bs2 = b_s2.reshape(1, -1)
    bb1 = b_b1.reshape(1, -1)
    bb2 = b_b2.reshape(1, -1)
    bv1 = b_v1.reshape(1, -1)
    bv2 = b_v2.reshape(1, -1)

    weights = (W_s1, bs1, W_s2, bs2, W_b1, bb1, W_b2, bb2,
               wv1a, wv1b, bv1, W_v2, bv2)

    out = pl.pallas_call(
        _tc_body,
        grid=(_NBLK,),
        in_specs=[
            pl.BlockSpec((1, 1, _BLK), lambda i: (i, 0, 0)),
            pl.BlockSpec((1, 1, _BLK), lambda i: (i, 0, 0)),
            pl.BlockSpec((1, 1, _BLK), lambda i: (i + _NBLK, 0, 0)),
            pl.BlockSpec((_BLK, 128), lambda i: (i, 0)),
            pl.BlockSpec((1, 1, _N_REGIONS), lambda i: (0, 0, 0)),
            pl.BlockSpec((_N_REGIONS, 128), lambda i: (0, 0)),
        ] + [_full_spec(w) for w in weights],
        out_specs=pl.BlockSpec((_B, 1), lambda i: (0, 0)),
        out_shape=jax.ShapeDtypeStruct((_B, 1), jnp.float32),
        scratch_shapes=[
            pltpu.VMEM((_B, 128), jnp.float32),
            pltpu.VMEM((_B, 1), jnp.float32),
        ],
    )(nbi3, parts3, parts3, node_embeddings, rbi3, region_embeddings, *weights)
    return out[:, 0]


def kernel(node_embeddings, region_embeddings, boundary_nodes,
           node_batch_idx, region_batch_idx, action_mask,
           W_s1, b_s1, W_s2, b_s2, W_b1, b_b1, W_b2, b_b2,
           W_v1, b_v1, W_v2, b_v2):
    del action_mask
    parts = _hist(boundary_nodes.astype(jnp.int32),
                  jnp.zeros((_N_NODES,), jnp.float32))
    return _tc_forward(parts, node_embeddings, region_embeddings,
                       node_batch_idx, region_batch_idx,
                       W_s1, b_s1, W_s2, b_s2, W_b1, b_b1, W_b2, b_b2,
                       W_v1, b_v1, W_v2, b_v2)
